# speedup vs baseline: 1.5515x; 1.5515x over previous
import functools

import jax
import jax.numpy as jnp
from jax import lax
from jax.experimental import pallas as pl
from jax.experimental.pallas import tpu as pltpu
from jax.experimental.pallas import tpu_sc as plsc

B, S, H = 4, 2048, 1024
T = B * S
OFFSET = 2

NW = 32
ROWS_PER_TILE = T // NW
CHUNK = 32
NCHUNK = ROWS_PER_TILE // CHUNK


def _sc_gather(word_emb, ids2d):
  mesh = plsc.VectorSubcoreMesh(core_axis_name="c", subcore_axis_name="s")

  @functools.partial(
      pl.kernel,
      mesh=mesh,
      out_type=jax.ShapeDtypeStruct((T, H), jnp.float32),
      scratch_types=[
          pltpu.VMEM((NCHUNK, CHUNK), jnp.int32),
          pltpu.VMEM((CHUNK, H), jnp.float32),
          pltpu.VMEM((CHUNK, H), jnp.float32),
          pltpu.SemaphoreType.DMA,
          pltpu.SemaphoreType.DMA,
          pltpu.SemaphoreType.DMA,
          pltpu.SemaphoreType.DMA,
      ],
  )
  def gather_kernel(table, idx, out, idx_v, buf0, buf1, g0, g1, o0, o1):
    wid = lax.axis_index("s") * 2 + lax.axis_index("c")
    chunk0 = wid * NCHUNK
    pltpu.sync_copy(idx.at[pl.ds(chunk0, NCHUNK)], idx_v)
    bufs = (buf0, buf1)
    gsems = (g0, g1)
    osems = (o0, o1)
    gcp = [None, None]
    ocp = [None, None]
    gcp[0] = pltpu.async_copy(table.at[idx_v.at[0]], bufs[0], gsems[0])
    for c in range(NCHUNK):
      b = c & 1
      nb = b ^ 1
      if c + 1 < NCHUNK:
        if c >= 1:
          ocp[nb].wait()
        gcp[nb] = pltpu.async_copy(table.at[idx_v.at[c + 1]], bufs[nb], gsems[nb])
      gcp[b].wait()
      row0 = (chunk0 + c) * CHUNK
      ocp[b] = pltpu.async_copy(bufs[b], out.at[pl.ds(row0, CHUNK)], osems[b])
    ocp[0].wait()
    ocp[1].wait()

  return gather_kernel(word_emb, ids2d)


TOK = 1024


def _tc_embed_ln_body(gath_ref, pos_ref, tt_id_ref, ut_id_ref, tt_ref, ut_ref,
                      gamma_ref, beta_ref, out_ref):
  t = tt_id_ref[...]
  u = ut_id_ref[...]
  oh_t = (lax.broadcasted_iota(jnp.int32, (TOK, 2), 1) == t).astype(jnp.float32)
  oh_u = (lax.broadcasted_iota(jnp.int32, (TOK, 8), 1) == u).astype(jnp.float32)
  tt_c = lax.dot_general(oh_t, tt_ref[...], (((1,), (0,)), ((), ())),
                         preferred_element_type=jnp.float32,
                         precision=lax.Precision.HIGHEST)
  ut_c = lax.dot_general(oh_u, ut_ref[...], (((1,), (0,)), ((), ())),
                         preferred_element_type=jnp.float32,
                         precision=lax.Precision.HIGHEST)
  x = gath_ref[...] + pos_ref[...] + tt_c + ut_c
  mu = jnp.mean(x, axis=-1, keepdims=True)
  xc = x - mu
  var = jnp.mean(xc * xc, axis=-1, keepdims=True)
  y = xc * lax.rsqrt(var + 1e-5)
  out_ref[...] = y * gamma_ref[...] + beta_ref[...]


def _tc_embed_ln(gath, pos_used, tt_ids, ut_ids, tt_emb, ut_emb, gamma2, beta2):
  nsh = S // TOK
  return pl.pallas_call(
      _tc_embed_ln_body,
      grid=(nsh, B),
      in_specs=[
          pl.BlockSpec((TOK, H), lambda sh, b: (b * nsh + sh, 0)),
          pl.BlockSpec((TOK, H), lambda sh, b: (sh, 0)),
          pl.BlockSpec((TOK, 1), lambda sh, b: (b * nsh + sh, 0)),
          pl.BlockSpec((TOK, 1), lambda sh, b: (b * nsh + sh, 0)),
          pl.BlockSpec((2, H), lambda sh, b: (0, 0)),
          pl.BlockSpec((8, H), lambda sh, b: (0, 0)),
          pl.BlockSpec((1, H), lambda sh, b: (0, 0)),
          pl.BlockSpec((1, H), lambda sh, b: (0, 0)),
      ],
      out_specs=pl.BlockSpec((TOK, H), lambda sh, b: (b * nsh + sh, 0)),
      out_shape=jax.ShapeDtypeStruct((T, H), jnp.float32),
  )(gath, pos_used, tt_ids, ut_ids, tt_emb, ut_emb, gamma2, beta2)


def kernel(input_ids, token_type_ids, user_type_ids, word_emb, pos_emb, tt_emb,
           ut_emb, gamma, beta):
  ids2d = input_ids.reshape(NW * NCHUNK, CHUNK).astype(jnp.int32)
  gath = _sc_gather(word_emb, ids2d)
  pos_used = lax.slice(pos_emb, (OFFSET, 0), (OFFSET + S, H))
  tt_ids = token_type_ids.reshape(T, 1).astype(jnp.int32)
  ut_ids = user_type_ids.reshape(T, 1).astype(jnp.int32)
  out = _tc_embed_ln(gath, pos_used, tt_ids, ut_ids, tt_emb, ut_emb,
                     gamma.reshape(1, H), beta.reshape(1, H))
  return out.reshape(B, S, H)


# speedup vs baseline: 2.2293x; 1.4368x over previous
import functools

import jax
import jax.numpy as jnp
from jax import lax
from jax.experimental import pallas as pl
from jax.experimental.pallas import tpu as pltpu
from jax.experimental.pallas import tpu_sc as plsc

B, S, H = 4, 2048, 1024
T = B * S
OFFSET = 2

NW = 32
ROWS_PER_TILE = T // NW
CHUNK = 32
NCHUNK = ROWS_PER_TILE // CHUNK


def _sc_gather(word_emb, ids2d):
  mesh = plsc.VectorSubcoreMesh(core_axis_name="c", subcore_axis_name="s")

  @functools.partial(
      pl.kernel,
      mesh=mesh,
      out_type=jax.ShapeDtypeStruct((T, H), jnp.float32),
      scratch_types=[
          pltpu.VMEM((NCHUNK, CHUNK), jnp.int32),
          pltpu.VMEM((CHUNK, H), jnp.float32),
          pltpu.VMEM((CHUNK, H), jnp.float32),
          pltpu.VMEM((CHUNK, H), jnp.float32),
          pltpu.SemaphoreType.DMA,
          pltpu.SemaphoreType.DMA,
          pltpu.SemaphoreType.DMA,
          pltpu.SemaphoreType.DMA,
          pltpu.SemaphoreType.DMA,
          pltpu.SemaphoreType.DMA,
      ],
  )
  def gather_kernel(table, idx, out, idx_v, buf0, buf1, buf2,
                    g0, g1, g2, o0, o1, o2):
    wid = lax.axis_index("s") * 2 + lax.axis_index("c")
    chunk0 = wid * NCHUNK
    pltpu.sync_copy(idx.at[pl.ds(chunk0, NCHUNK)], idx_v)
    bufs = (buf0, buf1, buf2)
    gsems = (g0, g1, g2)
    osems = (o0, o1, o2)
    nbuf = len(bufs)
    gcp = [None] * nbuf
    ocp = [None] * nbuf
    for c in range(min(nbuf - 1, NCHUNK)):
      gcp[c] = pltpu.async_copy(table.at[idx_v.at[c]], bufs[c], gsems[c])
    for c in range(NCHUNK):
      b = c % nbuf
      pc = c + nbuf - 1
      if pc < NCHUNK:
        pb = pc % nbuf
        if ocp[pb] is not None:
          ocp[pb].wait()
        gcp[pb] = pltpu.async_copy(table.at[idx_v.at[pc]], bufs[pb], gsems[pb])
      gcp[b].wait()
      row0 = (chunk0 + c) * CHUNK
      ocp[b] = pltpu.async_copy(bufs[b], out.at[pl.ds(row0, CHUNK)], osems[b])
    for b in range(min(nbuf, NCHUNK)):
      ocp[b].wait()

  return gather_kernel(word_emb, ids2d)


TOK = 1024


def _tc_embed_ln_body(gath_ref, pos_ref, tt_id_ref, ut_id_ref, tt_ref, ut_ref,
                      gamma_ref, beta_ref, out_ref):
  t = tt_id_ref[...]
  u = ut_id_ref[...]
  oh_t = (lax.broadcasted_iota(jnp.int32, (TOK, 2), 1) == t).astype(jnp.float32)
  oh_u = (lax.broadcasted_iota(jnp.int32, (TOK, 8), 1) == u).astype(jnp.float32)
  tt_c = lax.dot_general(oh_t, tt_ref[...], (((1,), (0,)), ((), ())),
                         preferred_element_type=jnp.float32,
                         precision=lax.Precision.DEFAULT)
  ut_c = lax.dot_general(oh_u, ut_ref[...], (((1,), (0,)), ((), ())),
                         preferred_element_type=jnp.float32,
                         precision=lax.Precision.DEFAULT)
  x = gath_ref[...] + pos_ref[...] + tt_c + ut_c
  mu = jnp.mean(x, axis=-1, keepdims=True)
  xc = x - mu
  var = jnp.mean(xc * xc, axis=-1, keepdims=True)
  y = xc * lax.rsqrt(var + 1e-5)
  out_ref[...] = y * gamma_ref[...] + beta_ref[...]


def _tc_embed_ln(gath, pos_used, tt_ids, ut_ids, tt_emb, ut_emb, gamma2, beta2):
  nsh = S // TOK
  return pl.pallas_call(
      _tc_embed_ln_body,
      grid=(nsh, B),
      in_specs=[
          pl.BlockSpec((TOK, H), lambda sh, b: (b * nsh + sh, 0)),
          pl.BlockSpec((TOK, H), lambda sh, b: (sh, 0)),
          pl.BlockSpec((TOK, 1), lambda sh, b: (b * nsh + sh, 0)),
          pl.BlockSpec((TOK, 1), lambda sh, b: (b * nsh + sh, 0)),
          pl.BlockSpec((2, H), lambda sh, b: (0, 0)),
          pl.BlockSpec((8, H), lambda sh, b: (0, 0)),
          pl.BlockSpec((1, H), lambda sh, b: (0, 0)),
          pl.BlockSpec((1, H), lambda sh, b: (0, 0)),
      ],
      out_specs=pl.BlockSpec((TOK, H), lambda sh, b: (b * nsh + sh, 0)),
      out_shape=jax.ShapeDtypeStruct((T, H), jnp.float32),
  )(gath, pos_used, tt_ids, ut_ids, tt_emb, ut_emb, gamma2, beta2)


def kernel(input_ids, token_type_ids, user_type_ids, word_emb, pos_emb, tt_emb,
           ut_emb, gamma, beta):
  ids2d = input_ids.reshape(NW * NCHUNK, CHUNK).astype(jnp.int32)
  gath = _sc_gather(word_emb, ids2d)
  pos_used = lax.slice(pos_emb, (OFFSET, 0), (OFFSET + S, H))
  tt_ids = token_type_ids.reshape(T, 1).astype(jnp.int32)
  ut_ids = user_type_ids.reshape(T, 1).astype(jnp.int32)
  out = _tc_embed_ln(gath, pos_used, tt_ids, ut_ids, tt_emb, ut_emb,
                     gamma.reshape(1, H), beta.reshape(1, H))
  return out.reshape(B, S, H)
